# Initial kernel scaffold; baseline (speedup 1.0000x reference)
import jax
import jax.numpy as jnp
from jax.experimental import pallas as pl


def kernel(x, types, edge_index, edge_label, small_idx, W_h0, b_h0, W_h1, b_h1, Wc1, bc1, Wc2, bc2, Wl0, bl0):
    raise NotImplementedError("write your pallas kernel here")


# speedup vs baseline: 21.0807x; 21.0807x over previous
import functools

import jax
import jax.numpy as jnp
from jax import lax
from jax.experimental import pallas as pl
from jax.experimental.pallas import tpu as pltpu
from jax.experimental.pallas import tpu_sc as plsc

_N, _E, _D = 10000, 320000, 128
_NC, _NS = 2, 16
_NW = _NC * _NS
_K = 125
_CPW = _E // (_K * _NW)
_RPT = _N // _NS
_DW = 16


def _sc_mesh():
    return plsc.VectorSubcoreMesh(
        core_axis_name="c", subcore_axis_name="s",
        num_cores=_NC, num_subcores=_NS)


_SC_PARAMS = pltpu.CompilerParams(use_tc_tiling_on_sc=False)


def _zero_fill(z_ref, rows, width):
    zero16 = jnp.zeros((16,), jnp.float32)
    for r in range(rows):
        for c in range(width // 16):
            z_ref[r, pl.ds(c * 16, 16)] = zero16


def _deg_partials(dst2d):
    @functools.partial(
        pl.kernel,
        out_type=jax.ShapeDtypeStruct((_NC, _N, _DW), jnp.float32),
        mesh=_sc_mesh(),
        scratch_types=[
            pltpu.VMEM((_CPW, _K), jnp.int32),
            pltpu.VMEM((_K, _DW), jnp.float32),
            pltpu.VMEM((_K, _DW), jnp.float32),
            pltpu.VMEM_SHARED((_N, _DW), jnp.float32),
        ],
        compiler_params=_SC_PARAMS,
    )
    def k(dst_hbm, out_hbm, dst_v, ones_v, z_v, acc):
        cid = lax.axis_index("c")
        sid = lax.axis_index("s")
        wid = cid * _NS + sid
        one16 = jnp.ones((16,), jnp.float32)
        for r in range(_K):
            ones_v[r, pl.ds(0, 16)] = one16
        _zero_fill(z_v, _K, _DW)
        row0 = sid * _RPT

        def zcopy(j, _):
            pltpu.sync_copy(z_v, acc.at[pl.ds(row0 + j * _K, _K)])
            return _

        lax.fori_loop(0, _RPT // _K, zcopy, 0)
        pltpu.sync_copy(dst_hbm.at[pl.ds(wid * _CPW, _CPW)], dst_v)
        plsc.subcore_barrier()

        def body(j, _):
            pltpu.sync_copy(ones_v, acc.at[dst_v.at[j]], add=True)
            return _

        lax.fori_loop(0, _CPW, body, 0)
        plsc.subcore_barrier()
        pltpu.sync_copy(acc.at[pl.ds(row0, _RPT)],
                        out_hbm.at[cid, pl.ds(row0, _RPT)])

    return k(dst2d)


def _edge_scatter(g, src2d, dst2d):
    @functools.partial(
        pl.kernel,
        out_type=jax.ShapeDtypeStruct((_NC, _N, _D), jnp.float32),
        mesh=_sc_mesh(),
        scratch_types=[
            pltpu.VMEM((_CPW, _K), jnp.int32),
            pltpu.VMEM((_CPW, _K), jnp.int32),
            pltpu.VMEM((_K, _D), jnp.float32),
            pltpu.VMEM((25, _D), jnp.float32),
            pltpu.VMEM_SHARED((_N, _D), jnp.float32),
            pltpu.SemaphoreType.DMA,
        ],
        compiler_params=_SC_PARAMS,
    )
    def k(g_hbm, src_hbm, dst_hbm, out_hbm, src_v, dst_v, rows_v, z_v, acc, sem):
        cid = lax.axis_index("c")
        sid = lax.axis_index("s")
        wid = cid * _NS + sid
        _zero_fill(z_v, 25, _D)
        row0 = sid * _RPT

        def zcopy(j, _):
            pltpu.sync_copy(z_v, acc.at[pl.ds(row0 + j * 25, 25)])
            return _

        lax.fori_loop(0, _RPT // 25, zcopy, 0)
        pltpu.sync_copy(src_hbm.at[pl.ds(wid * _CPW, _CPW)], src_v)
        pltpu.sync_copy(dst_hbm.at[pl.ds(wid * _CPW, _CPW)], dst_v)
        plsc.subcore_barrier()

        def body(j, _):
            pltpu.async_copy(g_hbm.at[src_v.at[j]], rows_v, sem).wait()
            pltpu.sync_copy(rows_v, acc.at[dst_v.at[j]], add=True)
            return _

        lax.fori_loop(0, _CPW, body, 0)
        plsc.subcore_barrier()
        pltpu.sync_copy(acc.at[pl.ds(row0, _RPT)],
                        out_hbm.at[cid, pl.ds(row0, _RPT)])

    return k(g, src2d, dst2d)


def _k1_input_transform(x, types, W_h0, b_h0, W_h1, b_h1):
    def body(x_ref, t_ref, w0_ref, b0_ref, w1_ref, b1_ref, o_ref):
        xv = x_ref[...]
        x1 = xv * t_ref[...]
        x0 = (xv - x1)[:, :64]
        a = jnp.maximum(jnp.dot(x0, w0_ref[...],
                                preferred_element_type=jnp.float32)
                        + b0_ref[...], 0.0)
        b = jnp.maximum(jnp.dot(x1, w1_ref[...],
                                preferred_element_type=jnp.float32)
                        + b1_ref[...], 0.0)
        o_ref[...] = a + b

    return pl.pallas_call(
        body, out_shape=jax.ShapeDtypeStruct((_N, _D), jnp.float32),
    )(x, types, W_h0, b_h0.reshape(1, -1), W_h1, b_h1.reshape(1, -1))


def _k2_first_g(h, Wc1, degp):
    def body(h_ref, w_ref, d_ref, g_ref, dinv_ref):
        deg = d_ref[0, :, 0:1] + d_ref[1, :, 0:1] + 1.0
        dinv = lax.rsqrt(deg)
        dinv_ref[...] = dinv
        g_ref[...] = jnp.dot(h_ref[...], w_ref[...],
                             preferred_element_type=jnp.float32) * dinv

    return pl.pallas_call(
        body,
        out_shape=(jax.ShapeDtypeStruct((_N, _D), jnp.float32),
                   jax.ShapeDtypeStruct((_N, 1), jnp.float32)),
    )(h, Wc1, degp)


def _k3_combine_g(s, g, dinv, b, W):
    def body(s_ref, g_ref, dinv_ref, b_ref, w_ref, o_ref):
        dinv = dinv_ref[...]
        h2 = jnp.maximum(dinv * (s_ref[0] + s_ref[1] + g_ref[...])
                         + b_ref[...], 0.0)
        o_ref[...] = jnp.dot(h2, w_ref[...],
                             preferred_element_type=jnp.float32) * dinv

    return pl.pallas_call(
        body, out_shape=jax.ShapeDtypeStruct((_N, _D), jnp.float32),
    )(s, g, dinv, b.reshape(1, -1), W)


def _k4_combine_head(s, g, dinv, b, Wl0, bl0):
    def body(s_ref, g_ref, dinv_ref, b_ref, w_ref, bl_ref, o_ref):
        h3 = jnp.maximum(dinv_ref[...] * (s_ref[0] + s_ref[1] + g_ref[...])
                         + b_ref[...], 0.0)
        o_ref[...] = jnp.maximum(
            jnp.dot(h3, w_ref[...], preferred_element_type=jnp.float32)
            + bl_ref[...], 0.0)

    return pl.pallas_call(
        body, out_shape=jax.ShapeDtypeStruct((_N, _D), jnp.float32),
    )(s, g, dinv, b.reshape(1, -1), Wl0, bl0.reshape(1, -1))


def kernel(x, types, edge_index, edge_label, small_idx,
           W_h0, b_h0, W_h1, b_h1, Wc1, bc1, Wc2, bc2, Wl0, bl0):
    src2d = edge_index[0].reshape(_E // _K, _K)
    dst2d = edge_index[1].reshape(_E // _K, _K)

    degp = _deg_partials(dst2d)
    h = _k1_input_transform(x, types, W_h0, b_h0, W_h1, b_h1)
    g1, dinv = _k2_first_g(h, Wc1, degp)
    s1 = _edge_scatter(g1, src2d, dst2d)
    g2 = _k3_combine_g(s1, g1, dinv, bc1, Wc2)
    s2 = _edge_scatter(g2, src2d, dst2d)
    return _k4_combine_head(s2, g2, dinv, bc2, Wl0, bl0)
